# baseline (speedup 1.0000x reference)
import jax
import jax.numpy as jnp
from jax.experimental import pallas as pl
from jax.experimental.pallas import tpu as pltpu
from jax.experimental.pallas import tpu_sc as plsc

EPS = 1e-3

_GATHER_WINDOW = 128
_ROW_BLOCK = 512


def _sc_gather(tok_emb, idx2d, n, d):
    mesh = plsc.VectorSubcoreMesh(core_axis_name="core",
                                  subcore_axis_name="subcore")

    @pl.kernel(out_type=jax.ShapeDtypeStruct((n, d), tok_emb.dtype),
               mesh=mesh)
    def gather_kernel(x_hbm, i_hbm, o_hbm):
        def body(i_vmem, o_vmem):
            pltpu.sync_copy(x_hbm.at[i_vmem.at[0]], o_vmem)

        pltpu.emit_pipeline(
            body,
            grid=(n // _GATHER_WINDOW,),
            in_specs=[pl.BlockSpec((1, _GATHER_WINDOW),
                                   index_map=lambda i: (0, i))],
            out_specs=[pl.BlockSpec((_GATHER_WINDOW, d),
                                    index_map=lambda i: (i, 0))],
            core_axis_name=("core", "subcore"),
            dimension_semantics=(pltpu.PARALLEL,),
        )(i_hbm, o_hbm)

    return gather_kernel(tok_emb, idx2d)


def _dense_body(x_ref, pos_ref, gamma_ref, beta_ref, w_ref, b_ref, o_ref):
    d = pos_ref.shape[1]
    x = x_ref[:, :d] + pos_ref[...]
    mean = jnp.mean(x, axis=1, keepdims=True)
    xc = x - mean
    var = jnp.mean(xc * xc, axis=1, keepdims=True)
    xn = xc * jax.lax.rsqrt(var + EPS)
    xn = xn * gamma_ref[...] + beta_ref[...]
    o_ref[...] = (
        jnp.dot(xn, w_ref[...], preferred_element_type=jnp.float32)
        + b_ref[...]
    )


def _tc_dense(x, pos_tiled, gamma, beta, W, b, n, d, v, interpret=False):
    grid = (n // _ROW_BLOCK,)
    dx = x.shape[1]
    return pl.pallas_call(
        _dense_body,
        grid=grid,
        in_specs=[
            pl.BlockSpec((_ROW_BLOCK, dx), lambda i: (i, 0)),
            pl.BlockSpec((_ROW_BLOCK, d), lambda i: (0, 0)),
            pl.BlockSpec((1, d), lambda i: (0, 0)),
            pl.BlockSpec((1, d), lambda i: (0, 0)),
            pl.BlockSpec((d, v), lambda i: (0, 0)),
            pl.BlockSpec((1, v), lambda i: (0, 0)),
        ],
        out_specs=pl.BlockSpec((_ROW_BLOCK, v), lambda i: (i, 0)),
        out_shape=jax.ShapeDtypeStruct((n, v), jnp.float32),
        compiler_params=pltpu.CompilerParams(
            dimension_semantics=("arbitrary",),
        ),
        interpret=interpret,
    )(x, pos_tiled, gamma, beta, W, b)


def kernel(inputs, tok_emb, pos_emb, gamma, beta, W, b):
    batch, seq = inputs.shape
    vocab, d = tok_emb.shape
    v_out = W.shape[1]
    n = batch * seq

    idx2d = inputs.reshape(1, n).astype(jnp.int32)
    d_pad = 128
    tok_pad = jnp.pad(tok_emb, ((0, 0), (0, d_pad - d)))
    x = _sc_gather(tok_pad, idx2d, n, d_pad)

    pos_tiled = jnp.tile(pos_emb, (_ROW_BLOCK // seq, 1))
    logits = _tc_dense(x, pos_tiled, gamma.reshape(1, d), beta.reshape(1, d),
                       W, b.reshape(1, v_out), n, d, v_out)
    return logits.reshape(batch, seq, v_out)
